# Initial kernel scaffold; baseline (speedup 1.0000x reference)
import jax
import jax.numpy as jnp
from jax.experimental import pallas as pl


def kernel(logits, top_k):
    raise NotImplementedError("write your pallas kernel here")


# speedup vs baseline: 49.1717x; 49.1717x over previous
import jax
import jax.numpy as jnp
from jax.experimental import pallas as pl

FILTER = -1e9
K = 64
TOP_P = 0.9
MIN_KEEP = 3.0


def _body(x_ref, filt_ref, lp_ref):
    x = x_ref[...]
    R, C = x.shape

    def step(j, carry):
        t, vals, cnts = carry
        masked = jnp.where(x < t, x, FILTER)
        v = jnp.max(masked, axis=1, keepdims=True)
        c = jnp.sum((x == v).astype(jnp.float32), axis=1, keepdims=True)
        upd = jax.lax.broadcasted_iota(jnp.int32, (1, K), 1) == j
        vals = jnp.where(upd, v, vals)
        cnts = jnp.where(upd, c, cnts)
        return v, vals, cnts

    t0 = jnp.full((R, 1), jnp.inf, jnp.float32)
    vals0 = jnp.full((R, K), FILTER, jnp.float32)
    cnts0 = jnp.zeros((R, K), jnp.float32)
    _, vals, cnts = jax.lax.fori_loop(0, K, step, (t0, vals0, cnts0))

    def psum(a):
        for sh in (1, 2, 4, 8, 16, 32):
            a = a + jnp.concatenate(
                [jnp.zeros((R, sh), a.dtype), a[:, : K - sh]], axis=1)
        return a

    v0 = vals[:, 0:1]
    w = cnts * jnp.exp(vals - v0)
    cumw = psum(w)
    ncum = psum(cnts)
    nprev = ncum - cnts
    sv = nprev < 64.0
    svf = sv.astype(jnp.float32)
    s_all = jnp.sum(w * svf, axis=1, keepdims=True)
    n_all = jnp.sum(cnts * svf, axis=1, keepdims=True)
    below = (cumw / s_all) <= TOP_P
    r = jnp.sum(cnts * svf * below.astype(jnp.float32), axis=1, keepdims=True)
    m = jnp.minimum(jnp.maximum(r + 1.0, MIN_KEEP), n_all)

    keep = sv & (nprev < m)
    thresh = jnp.min(jnp.where(keep, vals, jnp.inf), axis=1, keepdims=True)
    s_kept = jnp.sum(w * keep.astype(jnp.float32), axis=1, keepdims=True)
    lse = v0 + jnp.log(s_kept)

    filt = jnp.where(x >= thresh, x, FILTER)
    filt_ref[...] = filt
    lp_ref[...] = filt - lse


def kernel(logits, top_k):
    del top_k
    B, V = logits.shape
    C = (V + 127) // 128 * 128
    x = jnp.pad(logits, ((0, 0), (0, C - V)), constant_values=FILTER)
    R = 8
    grid = (B // R,)
    spec = pl.BlockSpec((R, C), lambda i: (i, 0))
    filt, lp = pl.pallas_call(
        _body,
        grid=grid,
        in_specs=[spec],
        out_specs=[spec, spec],
        out_shape=[
            jax.ShapeDtypeStruct((B, C), jnp.float32),
            jax.ShapeDtypeStruct((B, C), jnp.float32),
        ],
    )(x)
    return filt[:, :V], lp[:, :V]


# speedup vs baseline: 225.9813x; 4.5958x over previous
import functools

import jax
import jax.numpy as jnp
from jax import lax
from jax.experimental import pallas as pl
from jax.experimental.pallas import tpu as pltpu
from jax.experimental.pallas import tpu_sc as plsc

FILTER = -1e9
K = 64
TOP_P = 0.9
MIN_KEEP = 3.0
NC = 2
NS = 16
ROWS_PER_TILE = 2


def _sortv(x):
    return jnp.sort(x)


def _rev(x):
    return jnp.flip(x, 0)


def _merge16(a, b):
    rb = _rev(b)
    return _sortv(jnp.minimum(a, rb)), _sortv(jnp.maximum(a, rb))


def _clean32(u, v):
    return _sortv(jnp.minimum(u, v)), _sortv(jnp.maximum(u, v))


def _merge32(a0, a1, b0, b1):
    r0, r1 = _rev(b1), _rev(b0)
    lo0, lo1 = jnp.minimum(a0, r0), jnp.minimum(a1, r1)
    hi0, hi1 = jnp.maximum(a0, r0), jnp.maximum(a1, r1)
    l0, l1 = _clean32(lo0, lo1)
    h0, h1 = _clean32(hi0, hi1)
    return l0, l1, h0, h1


def _clean64(h0, h1, h2, h3):
    a0, a1 = jnp.minimum(h0, h2), jnp.minimum(h1, h3)
    b0, b1 = jnp.maximum(h0, h2), jnp.maximum(h1, h3)
    l0, l1 = _clean32(a0, a1)
    u0, u1 = _clean32(b0, b1)
    return l0, l1, u0, u1


def _sc_body(x_hbm, out_hbm, row_v, pend_v, out_v):
    wid = lax.axis_index("s") * NC + lax.axis_index("c")
    C = x_hbm.shape[1]
    ngroups = C // 64

    def one_row(row):
        pltpu.sync_copy(x_hbm.at[row], row_v)

        def flush(carry):
            t, pc, t0, t1, t2, t3 = carry
            p0 = _sortv(pend_v[0:16])
            p1 = _sortv(pend_v[16:32])
            p2 = _sortv(pend_v[32:48])
            p3 = _sortv(pend_v[48:64])
            l0, h0 = _merge16(p0, p1)
            l1, h1 = _merge16(p2, p3)
            q0, q1, q2, q3 = _merge32(l0, h0, l1, h1)
            r0, r1, r2, r3 = _rev(q3), _rev(q2), _rev(q1), _rev(q0)
            n0, n1, n2, n3 = _clean64(
                jnp.maximum(t0, r0), jnp.maximum(t1, r1),
                jnp.maximum(t2, r2), jnp.maximum(t3, r3))
            sent = jnp.full((16,), FILTER, jnp.float32)
            pend_v[0:16] = sent
            pend_v[16:32] = sent
            pend_v[32:48] = sent
            pend_v[48:64] = sent
            nt = jnp.min(n0)
            return nt, jnp.int32(0), n0, n1, n2, n3

        def group(i, carry):
            t, pc, t0, t1, t2, t3 = carry
            base = i * 64
            x0 = row_v[pl.ds(base, 16)]
            x1 = row_v[pl.ds(base + 16, 16)]
            x2 = row_v[pl.ds(base + 32, 16)]
            x3 = row_v[pl.ds(base + 48, 16)]
            gmax = jnp.max(jnp.maximum(jnp.maximum(x0, x1),
                                       jnp.maximum(x2, x3)))

            def has_candidates(carry):
                t, pc, t0, t1, t2, t3 = carry
                for xx in (x0, x1, x2, x3):
                    mask = xx > t
                    plsc.store_compressed(pend_v.at[pl.ds(pc, 16)], xx, mask=mask)
                    pc = pc + jnp.sum(mask.astype(jnp.int32))
                    carry2 = (t, pc, t0, t1, t2, t3)
                    t, pc, t0, t1, t2, t3 = lax.cond(
                        pc >= 48, flush, lambda c: c, carry2)
                return t, pc, t0, t1, t2, t3

            return lax.cond(gmax > t, has_candidates, lambda c: c, carry)

        sent = jnp.full((16,), FILTER, jnp.float32)
        for off in range(0, 64, 16):
            pend_v[off:off + 16] = sent
        init = (jnp.float32(FILTER), jnp.int32(0), sent, sent, sent, sent)
        carry = lax.fori_loop(0, ngroups, group, init)
        _, _, t0, t1, t2, t3 = flush(carry)
        out_v[0:16] = _rev(t3)
        out_v[16:32] = _rev(t2)
        out_v[32:48] = _rev(t1)
        out_v[48:64] = _rev(t0)
        pltpu.sync_copy(out_v, out_hbm.at[row])

    for rr in range(ROWS_PER_TILE):
        one_row(wid * ROWS_PER_TILE + rr)


def _sc_topk(x):
    B, C = x.shape
    fn = functools.partial(
        pl.kernel,
        mesh=plsc.VectorSubcoreMesh(core_axis_name="c", subcore_axis_name="s"),
        compiler_params=pltpu.CompilerParams(needs_layout_passes=False),
        out_type=jax.ShapeDtypeStruct((B, K), jnp.float32),
        scratch_types=[
            pltpu.VMEM((C,), jnp.float32),
            pltpu.VMEM((K,), jnp.float32),
            pltpu.VMEM((K,), jnp.float32),
        ],
    )(_sc_body)
    return fn(x)


def _tc_body(x_ref, tv_ref, filt_ref, lp_ref):
    x = x_ref[...]
    tv = tv_ref[...]
    R = tv.shape[0]

    def psum(a):
        for sh in (1, 2, 4, 8, 16, 32):
            a = a + jnp.concatenate(
                [jnp.zeros((R, sh), a.dtype), a[:, : K - sh]], axis=1)
        return a

    v0 = tv[:, 0:1]
    w = jnp.exp(tv - v0)
    cumw = psum(w)
    s_all = cumw[:, K - 1:K]
    below = (cumw / s_all) <= TOP_P
    r = jnp.sum(below.astype(jnp.float32), axis=1, keepdims=True)
    m = jnp.maximum(r + 1.0, MIN_KEEP)
    pos = jax.lax.broadcasted_iota(jnp.int32, (1, K), 1).astype(jnp.float32)
    keepmask = pos < m
    thresh = jnp.min(jnp.where(keepmask, tv, jnp.inf), axis=1, keepdims=True)
    s_kept = jnp.sum(w * keepmask.astype(jnp.float32), axis=1, keepdims=True)
    lse = v0 + jnp.log(s_kept)
    filt = jnp.where(x >= thresh, x, FILTER)
    filt_ref[...] = filt
    lp_ref[...] = filt - lse


def kernel(logits, top_k):
    del top_k
    B, V = logits.shape
    C = (V + 63) // 64 * 64
    C = (C + 127) // 128 * 128
    x = jnp.pad(logits, ((0, 0), (0, C - V)), constant_values=FILTER)
    tv = _sc_topk(x)
    R = 8
    filt, lp = pl.pallas_call(
        _tc_body,
        grid=(B // R,),
        in_specs=[
            pl.BlockSpec((R, C), lambda i: (i, 0)),
            pl.BlockSpec((R, K), lambda i: (i, 0)),
        ],
        out_specs=[
            pl.BlockSpec((R, C), lambda i: (i, 0)),
            pl.BlockSpec((R, C), lambda i: (i, 0)),
        ],
        out_shape=[
            jax.ShapeDtypeStruct((B, C), jnp.float32),
            jax.ShapeDtypeStruct((B, C), jnp.float32),
        ],
    )(x, tv)
    return filt[:, :V], lp[:, :V]


# speedup vs baseline: 260.5848x; 1.1531x over previous
import functools

import jax
import jax.numpy as jnp
from jax import lax
from jax.experimental import pallas as pl
from jax.experimental.pallas import tpu as pltpu
from jax.experimental.pallas import tpu_sc as plsc

FILTER = -1e9
K = 64
TOP_P = 0.9
MIN_KEEP = 3.0
NC = 2
NS = 16
ROWS_PER_TILE = 2


def _sortv(x):
    return jnp.sort(x)


def _rev(x):
    return jnp.flip(x, 0)


def _merge16(a, b):
    rb = _rev(b)
    return _sortv(jnp.minimum(a, rb)), _sortv(jnp.maximum(a, rb))


def _clean32(u, v):
    return _sortv(jnp.minimum(u, v)), _sortv(jnp.maximum(u, v))


def _merge32(a0, a1, b0, b1):
    r0, r1 = _rev(b1), _rev(b0)
    lo0, lo1 = jnp.minimum(a0, r0), jnp.minimum(a1, r1)
    hi0, hi1 = jnp.maximum(a0, r0), jnp.maximum(a1, r1)
    l0, l1 = _clean32(lo0, lo1)
    h0, h1 = _clean32(hi0, hi1)
    return l0, l1, h0, h1


def _clean64(h0, h1, h2, h3):
    a0, a1 = jnp.minimum(h0, h2), jnp.minimum(h1, h3)
    b0, b1 = jnp.maximum(h0, h2), jnp.maximum(h1, h3)
    l0, l1 = _clean32(a0, a1)
    u0, u1 = _clean32(b0, b1)
    return l0, l1, u0, u1


def _sc_body(x_hbm, out_hbm, row_v, pend_v, out_v):
    wid = lax.axis_index("s") * NC + lax.axis_index("c")
    C = x_hbm.shape[1]
    ngroups = C // 128

    def one_row(row):
        pltpu.sync_copy(x_hbm.at[row], row_v)

        def flush(carry):
            t, pc, t0, t1, t2, t3 = carry
            p0 = _sortv(pend_v[0:16])
            p1 = _sortv(pend_v[16:32])
            p2 = _sortv(pend_v[32:48])
            p3 = _sortv(pend_v[48:64])
            l0, h0 = _merge16(p0, p1)
            l1, h1 = _merge16(p2, p3)
            q0, q1, q2, q3 = _merge32(l0, h0, l1, h1)
            r0, r1, r2, r3 = _rev(q3), _rev(q2), _rev(q1), _rev(q0)
            n0, n1, n2, n3 = _clean64(
                jnp.maximum(t0, r0), jnp.maximum(t1, r1),
                jnp.maximum(t2, r2), jnp.maximum(t3, r3))
            sent = jnp.full((16,), FILTER, jnp.float32)
            pend_v[0:16] = sent
            pend_v[16:32] = sent
            pend_v[32:48] = sent
            pend_v[48:64] = sent
            nt = jnp.min(n0)
            return nt, jnp.int32(0), n0, n1, n2, n3

        def group(i, carry):
            t, pc, t0, t1, t2, t3 = carry
            base = i * 128
            xs = [row_v[pl.ds(base + 16 * j, 16)] for j in range(8)]
            mx = xs[0]
            for xx in xs[1:]:
                mx = jnp.maximum(mx, xx)
            anyc = plsc.all_reduce_population_count(mx > t)[0]

            def has_candidates(carry):
                t, pc, t0, t1, t2, t3 = carry
                for xx in xs:
                    mask = xx > t
                    plsc.store_compressed(pend_v.at[pl.ds(pc, 16)], xx, mask=mask)
                    pc = pc + plsc.all_reduce_population_count(mask)[0]
                    carry2 = (t, pc, t0, t1, t2, t3)
                    t, pc, t0, t1, t2, t3 = lax.cond(
                        pc >= 48, flush, lambda c: c, carry2)
                return t, pc, t0, t1, t2, t3

            return lax.cond(anyc > 0, has_candidates, lambda c: c, carry)

        sent = jnp.full((16,), FILTER, jnp.float32)
        for off in range(0, 64, 16):
            pend_v[off:off + 16] = sent
        init = (jnp.float32(FILTER), jnp.int32(0), sent, sent, sent, sent)
        carry = lax.fori_loop(0, ngroups, group, init)
        _, _, t0, t1, t2, t3 = flush(carry)
        out_v[0:16] = _rev(t3)
        out_v[16:32] = _rev(t2)
        out_v[32:48] = _rev(t1)
        out_v[48:64] = _rev(t0)
        pltpu.sync_copy(out_v, out_hbm.at[row])

    for rr in range(ROWS_PER_TILE):
        one_row(wid * ROWS_PER_TILE + rr)


def _sc_topk(x):
    B, C = x.shape
    fn = functools.partial(
        pl.kernel,
        mesh=plsc.VectorSubcoreMesh(core_axis_name="c", subcore_axis_name="s"),
        compiler_params=pltpu.CompilerParams(needs_layout_passes=False),
        out_type=jax.ShapeDtypeStruct((B, K), jnp.float32),
        scratch_types=[
            pltpu.VMEM((C,), jnp.float32),
            pltpu.VMEM((K,), jnp.float32),
            pltpu.VMEM((K,), jnp.float32),
        ],
    )(_sc_body)
    return fn(x)


def _tc_body(x_ref, tv_ref, filt_ref, lp_ref):
    x = x_ref[...]
    tv = tv_ref[...]
    R = tv.shape[0]

    def psum(a):
        for sh in (1, 2, 4, 8, 16, 32):
            a = a + jnp.concatenate(
                [jnp.zeros((R, sh), a.dtype), a[:, : K - sh]], axis=1)
        return a

    v0 = tv[:, 0:1]
    w = jnp.exp(tv - v0)
    cumw = psum(w)
    s_all = cumw[:, K - 1:K]
    below = (cumw / s_all) <= TOP_P
    r = jnp.sum(below.astype(jnp.float32), axis=1, keepdims=True)
    m = jnp.maximum(r + 1.0, MIN_KEEP)
    pos = jax.lax.broadcasted_iota(jnp.int32, (1, K), 1).astype(jnp.float32)
    keepmask = pos < m
    thresh = jnp.min(jnp.where(keepmask, tv, jnp.inf), axis=1, keepdims=True)
    s_kept = jnp.sum(w * keepmask.astype(jnp.float32), axis=1, keepdims=True)
    lse = v0 + jnp.log(s_kept)
    filt = jnp.where(x >= thresh, x, FILTER)
    filt_ref[...] = filt
    lp_ref[...] = filt - lse


def kernel(logits, top_k):
    del top_k
    B, V = logits.shape
    C = (V + 63) // 64 * 64
    C = (C + 127) // 128 * 128
    x = jnp.pad(logits, ((0, 0), (0, C - V)), constant_values=FILTER)
    tv = _sc_topk(x)
    R = 8
    filt, lp = pl.pallas_call(
        _tc_body,
        grid=(B // R,),
        in_specs=[
            pl.BlockSpec((R, C), lambda i: (i, 0)),
            pl.BlockSpec((R, K), lambda i: (i, 0)),
        ],
        out_specs=[
            pl.BlockSpec((R, C), lambda i: (i, 0)),
            pl.BlockSpec((R, C), lambda i: (i, 0)),
        ],
        out_shape=[
            jax.ShapeDtypeStruct((B, C), jnp.float32),
            jax.ShapeDtypeStruct((B, C), jnp.float32),
        ],
    )(x, tv)
    return filt[:, :V], lp[:, :V]


# speedup vs baseline: 328.1629x; 1.2593x over previous
import functools

import jax
import jax.numpy as jnp
from jax import lax
from jax.experimental import pallas as pl
from jax.experimental.pallas import tpu as pltpu
from jax.experimental.pallas import tpu_sc as plsc

FILTER = -1e9
K = 64
TOP_P = 0.9
MIN_KEEP = 3.0
NC = 2
NS = 16
ROWS_PER_TILE = 2


def _sortv(x):
    return jnp.sort(x)


def _rev(x):
    return jnp.flip(x, 0)


def _merge16(a, b):
    rb = _rev(b)
    return _sortv(jnp.minimum(a, rb)), _sortv(jnp.maximum(a, rb))


def _clean32(u, v):
    return _sortv(jnp.minimum(u, v)), _sortv(jnp.maximum(u, v))


def _merge32(a0, a1, b0, b1):
    r0, r1 = _rev(b1), _rev(b0)
    lo0, lo1 = jnp.minimum(a0, r0), jnp.minimum(a1, r1)
    hi0, hi1 = jnp.maximum(a0, r0), jnp.maximum(a1, r1)
    l0, l1 = _clean32(lo0, lo1)
    h0, h1 = _clean32(hi0, hi1)
    return l0, l1, h0, h1


def _clean64(h0, h1, h2, h3):
    a0, a1 = jnp.minimum(h0, h2), jnp.minimum(h1, h3)
    b0, b1 = jnp.maximum(h0, h2), jnp.maximum(h1, h3)
    l0, l1 = _clean32(a0, a1)
    u0, u1 = _clean32(b0, b1)
    return l0, l1, u0, u1


def _sc_body(x_hbm, out_hbm, row_v, pend_v, out_v):
    wid = lax.axis_index("s") * NC + lax.axis_index("c")
    C = x_hbm.shape[1]
    ngroups = C // 128

    def one_row(row):
        pltpu.sync_copy(x_hbm.at[row], row_v)

        def reset_pend():
            sent = jnp.full((16,), FILTER, jnp.float32)
            for off in range(0, 192, 16):
                pend_v[off:off + 16] = sent

        def make_flush(nb):
            def fl(carry):
                t, pc, t0, t1, t2, t3 = carry
                for b in range(nb):
                    o = b * 64
                    p0 = _sortv(pend_v[o:o + 16])
                    p1 = _sortv(pend_v[o + 16:o + 32])
                    p2 = _sortv(pend_v[o + 32:o + 48])
                    p3 = _sortv(pend_v[o + 48:o + 64])
                    l0, h0 = _merge16(p0, p1)
                    l1, h1 = _merge16(p2, p3)
                    q0, q1, q2, q3 = _merge32(l0, h0, l1, h1)
                    r0, r1, r2, r3 = _rev(q3), _rev(q2), _rev(q1), _rev(q0)
                    t0, t1, t2, t3 = _clean64(
                        jnp.maximum(t0, r0), jnp.maximum(t1, r1),
                        jnp.maximum(t2, r2), jnp.maximum(t3, r3))
                reset_pend()
                return jnp.min(t0), jnp.int32(0), t0, t1, t2, t3
            return fl

        def tiered_flush(carry):
            def two_or_three(c):
                return lax.cond(c[1] <= 128, make_flush(2), make_flush(3), c)
            return lax.cond(carry[1] <= 64, make_flush(1), two_or_three, carry)

        def group(i, carry):
            t, pc, t0, t1, t2, t3 = carry
            base = i * 128
            xs = [row_v[pl.ds(base + 16 * j, 16)] for j in range(8)]
            mx = xs[0]
            for xx in xs[1:]:
                mx = jnp.maximum(mx, xx)
            anyc = plsc.all_reduce_population_count(mx > t)[0]

            def has_candidates(carry):
                t, pc, t0, t1, t2, t3 = carry
                for xx in xs:
                    mask = xx > t
                    plsc.store_compressed(pend_v.at[pl.ds(pc, 16)], xx, mask=mask)
                    pc = pc + plsc.all_reduce_population_count(mask)[0]
                carry2 = (t, pc, t0, t1, t2, t3)
                return lax.cond(pc >= 48, tiered_flush, lambda c: c, carry2)

            return lax.cond(anyc > 0, has_candidates, lambda c: c, carry)

        sent = jnp.full((16,), FILTER, jnp.float32)
        reset_pend()
        init = (jnp.float32(FILTER), jnp.int32(0), sent, sent, sent, sent)
        carry = lax.fori_loop(0, ngroups, group, init)
        _, _, t0, t1, t2, t3 = make_flush(3)(carry)
        out_v[0:16] = _rev(t3)
        out_v[16:32] = _rev(t2)
        out_v[32:48] = _rev(t1)
        out_v[48:64] = _rev(t0)
        pltpu.sync_copy(out_v, out_hbm.at[row])

    for rr in range(ROWS_PER_TILE):
        one_row(wid * ROWS_PER_TILE + rr)


def _sc_topk(x):
    B, C = x.shape
    fn = functools.partial(
        pl.kernel,
        mesh=plsc.VectorSubcoreMesh(core_axis_name="c", subcore_axis_name="s"),
        compiler_params=pltpu.CompilerParams(needs_layout_passes=False),
        out_type=jax.ShapeDtypeStruct((B, K), jnp.float32),
        scratch_types=[
            pltpu.VMEM((C,), jnp.float32),
            pltpu.VMEM((192,), jnp.float32),
            pltpu.VMEM((K,), jnp.float32),
        ],
    )(_sc_body)
    return fn(x)


def _tc_body(x_ref, tv_ref, filt_ref, lp_ref):
    x = x_ref[...]
    tv = tv_ref[...]
    R = tv.shape[0]

    def psum(a):
        for sh in (1, 2, 4, 8, 16, 32):
            a = a + jnp.concatenate(
                [jnp.zeros((R, sh), a.dtype), a[:, : K - sh]], axis=1)
        return a

    v0 = tv[:, 0:1]
    w = jnp.exp(tv - v0)
    cumw = psum(w)
    s_all = cumw[:, K - 1:K]
    below = (cumw / s_all) <= TOP_P
    r = jnp.sum(below.astype(jnp.float32), axis=1, keepdims=True)
    m = jnp.maximum(r + 1.0, MIN_KEEP)
    pos = jax.lax.broadcasted_iota(jnp.int32, (1, K), 1).astype(jnp.float32)
    keepmask = pos < m
    thresh = jnp.min(jnp.where(keepmask, tv, jnp.inf), axis=1, keepdims=True)
    s_kept = jnp.sum(w * keepmask.astype(jnp.float32), axis=1, keepdims=True)
    lse = v0 + jnp.log(s_kept)
    filt = jnp.where(x >= thresh, x, FILTER)
    filt_ref[...] = filt
    lp_ref[...] = filt - lse


def kernel(logits, top_k):
    del top_k
    B, V = logits.shape
    C = (V + 63) // 64 * 64
    C = (C + 127) // 128 * 128
    x = jnp.pad(logits, ((0, 0), (0, C - V)), constant_values=FILTER)
    tv = _sc_topk(x)
    R = 8
    filt, lp = pl.pallas_call(
        _tc_body,
        grid=(B // R,),
        in_specs=[
            pl.BlockSpec((R, C), lambda i: (i, 0)),
            pl.BlockSpec((R, K), lambda i: (i, 0)),
        ],
        out_specs=[
            pl.BlockSpec((R, C), lambda i: (i, 0)),
            pl.BlockSpec((R, C), lambda i: (i, 0)),
        ],
        out_shape=[
            jax.ShapeDtypeStruct((B, C), jnp.float32),
            jax.ShapeDtypeStruct((B, C), jnp.float32),
        ],
    )(x, tv)
    return filt[:, :V], lp[:, :V]


# speedup vs baseline: 415.6084x; 1.2665x over previous
import functools

import jax
import jax.numpy as jnp
from jax import lax
from jax.experimental import pallas as pl
from jax.experimental.pallas import tpu as pltpu
from jax.experimental.pallas import tpu_sc as plsc

FILTER = -1e9
K = 64
TOP_P = 0.9
MIN_KEEP = 3.0
NC = 2
NS = 16
ROWS_PER_TILE = 2


def _sortv(x):
    return jnp.sort(x)


def _rev(x):
    return jnp.flip(x, 0)


def _merge16(a, b):
    rb = _rev(b)
    return _sortv(jnp.minimum(a, rb)), _sortv(jnp.maximum(a, rb))


def _clean32(u, v):
    return _sortv(jnp.minimum(u, v)), _sortv(jnp.maximum(u, v))


def _merge32(a0, a1, b0, b1):
    r0, r1 = _rev(b1), _rev(b0)
    lo0, lo1 = jnp.minimum(a0, r0), jnp.minimum(a1, r1)
    hi0, hi1 = jnp.maximum(a0, r0), jnp.maximum(a1, r1)
    l0, l1 = _clean32(lo0, lo1)
    h0, h1 = _clean32(hi0, hi1)
    return l0, l1, h0, h1


def _clean64(h0, h1, h2, h3):
    a0, a1 = jnp.minimum(h0, h2), jnp.minimum(h1, h3)
    b0, b1 = jnp.maximum(h0, h2), jnp.maximum(h1, h3)
    l0, l1 = _clean32(a0, a1)
    u0, u1 = _clean32(b0, b1)
    return l0, l1, u0, u1


def _sc_body(x_hbm, out_hbm, row_v, pend_v, out_v):
    wid = lax.axis_index("s") * NC + lax.axis_index("c")
    C = x_hbm.shape[1]
    ngroups = C // 256

    def one_row(row):
        pltpu.sync_copy(x_hbm.at[row], row_v)

        def reset_pend():
            sent = jnp.full((16,), FILTER, jnp.float32)
            for off in range(0, 192, 16):
                pend_v[off:off + 16] = sent

        def make_flush(nb):
            def fl(carry):
                t, pc, t0, t1, t2, t3 = carry
                for b in range(nb):
                    o = b * 64
                    p0 = _sortv(pend_v[o:o + 16])
                    p1 = _sortv(pend_v[o + 16:o + 32])
                    p2 = _sortv(pend_v[o + 32:o + 48])
                    p3 = _sortv(pend_v[o + 48:o + 64])
                    l0, h0 = _merge16(p0, p1)
                    l1, h1 = _merge16(p2, p3)
                    q0, q1, q2, q3 = _merge32(l0, h0, l1, h1)
                    r0, r1, r2, r3 = _rev(q3), _rev(q2), _rev(q1), _rev(q0)
                    t0, t1, t2, t3 = _clean64(
                        jnp.maximum(t0, r0), jnp.maximum(t1, r1),
                        jnp.maximum(t2, r2), jnp.maximum(t3, r3))
                reset_pend()
                return jnp.min(t0), jnp.int32(0), t0, t1, t2, t3
            return fl

        def tiered_flush(carry):
            def two_or_three(c):
                return lax.cond(c[1] <= 128, make_flush(2), make_flush(3), c)
            return lax.cond(carry[1] <= 64, make_flush(1), two_or_three, carry)

        def group(i, carry):
            t, pc, t0, t1, t2, t3 = carry
            base = i * 256
            xs = [row_v[pl.ds(base + 16 * j, 16)] for j in range(16)]
            mx = xs[0]
            for xx in xs[1:]:
                mx = jnp.maximum(mx, xx)
            anyc = plsc.all_reduce_population_count(mx > t)[0]

            def has_candidates(carry):
                t, pc, t0, t1, t2, t3 = carry
                for half in (xs[:8], xs[8:]):
                    for xx in half:
                        mask = xx > t
                        plsc.store_compressed(
                            pend_v.at[pl.ds(pc, 16)], xx, mask=mask)
                        pc = pc + plsc.all_reduce_population_count(mask)[0]
                    carry2 = (t, pc, t0, t1, t2, t3)
                    t, pc, t0, t1, t2, t3 = lax.cond(
                        pc >= 48, tiered_flush, lambda c: c, carry2)
                return t, pc, t0, t1, t2, t3

            return lax.cond(anyc > 0, has_candidates, lambda c: c, carry)

        sent = jnp.full((16,), FILTER, jnp.float32)
        reset_pend()
        init = (jnp.float32(FILTER), jnp.int32(0), sent, sent, sent, sent)
        carry = lax.fori_loop(0, ngroups, group, init)
        _, _, t0, t1, t2, t3 = make_flush(3)(carry)
        out_v[0:16] = _rev(t3)
        out_v[16:32] = _rev(t2)
        out_v[32:48] = _rev(t1)
        out_v[48:64] = _rev(t0)
        pltpu.sync_copy(out_v, out_hbm.at[row])

    for rr in range(ROWS_PER_TILE):
        one_row(wid * ROWS_PER_TILE + rr)


def _sc_topk(x):
    B, C = x.shape
    fn = functools.partial(
        pl.kernel,
        mesh=plsc.VectorSubcoreMesh(core_axis_name="c", subcore_axis_name="s"),
        compiler_params=pltpu.CompilerParams(needs_layout_passes=False),
        out_type=jax.ShapeDtypeStruct((B, K), jnp.float32),
        scratch_types=[
            pltpu.VMEM((C,), jnp.float32),
            pltpu.VMEM((192,), jnp.float32),
            pltpu.VMEM((K,), jnp.float32),
        ],
    )(_sc_body)
    return fn(x)


def _tc_body(x_ref, tv_ref, filt_ref, lp_ref):
    x = x_ref[...]
    tv = tv_ref[...]
    R = tv.shape[0]

    def psum(a):
        for sh in (1, 2, 4, 8, 16, 32):
            a = a + jnp.concatenate(
                [jnp.zeros((R, sh), a.dtype), a[:, : K - sh]], axis=1)
        return a

    v0 = tv[:, 0:1]
    w = jnp.exp(tv - v0)
    cumw = psum(w)
    s_all = cumw[:, K - 1:K]
    below = (cumw / s_all) <= TOP_P
    r = jnp.sum(below.astype(jnp.float32), axis=1, keepdims=True)
    m = jnp.maximum(r + 1.0, MIN_KEEP)
    pos = jax.lax.broadcasted_iota(jnp.int32, (1, K), 1).astype(jnp.float32)
    keepmask = pos < m
    thresh = jnp.min(jnp.where(keepmask, tv, jnp.inf), axis=1, keepdims=True)
    s_kept = jnp.sum(w * keepmask.astype(jnp.float32), axis=1, keepdims=True)
    lse = v0 + jnp.log(s_kept)
    filt = jnp.where(x >= thresh, x, FILTER)
    filt_ref[...] = filt
    lp_ref[...] = filt - lse


def kernel(logits, top_k):
    del top_k
    B, V = logits.shape
    C = (V + 255) // 256 * 256
    x = jnp.pad(logits, ((0, 0), (0, C - V)), constant_values=FILTER)
    tv = _sc_topk(x)
    R = 8
    filt, lp = pl.pallas_call(
        _tc_body,
        grid=(B // R,),
        in_specs=[
            pl.BlockSpec((R, C), lambda i: (i, 0)),
            pl.BlockSpec((R, K), lambda i: (i, 0)),
        ],
        out_specs=[
            pl.BlockSpec((R, C), lambda i: (i, 0)),
            pl.BlockSpec((R, C), lambda i: (i, 0)),
        ],
        out_shape=[
            jax.ShapeDtypeStruct((B, C), jnp.float32),
            jax.ShapeDtypeStruct((B, C), jnp.float32),
        ],
    )(x, tv)
    return filt[:, :V], lp[:, :V]
